# Initial kernel scaffold; baseline (speedup 1.0000x reference)
import jax
import jax.numpy as jnp
from jax.experimental import pallas as pl


def kernel(x, edge_attr, edge_index, batch, lin1_w, lin1_b, conv_fc_w, conv_fc_b, conv_bn1_g, conv_bn1_b, conv_bn2_g, conv_bn2_b, lin2_w, lin2_b, bn_in_g, bn_in_b, cl1_w, cl1_b, cbn1_g, cbn1_b, cl2_w, cl2_b, cbn2_g, cbn2_b, reg_w, reg_b):
    raise NotImplementedError("write your pallas kernel here")


# speedup vs baseline: 1.5492x; 1.5492x over previous
import functools

import jax
import jax.numpy as jnp
from jax import lax
from jax.experimental import pallas as pl
from jax.experimental.pallas import tpu as pltpu
from jax.experimental.pallas import tpu_sc as plsc

_N = 50000
_M = 16
_E = 800000
_G = 256
_AF = 64
_NF = 41
_ED = 128
_NCONV = 3
_EPS = 1e-5

_TA = 200
_GRID = _N // _TA
_TN = 1000


_SC_CHUNK = 128
_SC_NBLK = _E // _SC_CHUNK
_SC_NW = 32
_SC_ITERS = (_SC_NBLK + _SC_NW - 1) // _SC_NW


def _sc_gather(table, idx):
    mesh = plsc.VectorSubcoreMesh(core_axis_name="c", subcore_axis_name="s")

    @functools.partial(
        pl.kernel,
        out_type=jax.ShapeDtypeStruct((_E, _AF), jnp.float32),
        mesh=mesh,
        scratch_types=[
            pltpu.VMEM((_SC_CHUNK,), jnp.int32),
            pltpu.VMEM((_SC_CHUNK, _AF), jnp.float32),
            pltpu.SemaphoreType.DMA,
        ],
        compiler_params=pltpu.CompilerParams(use_tc_tiling_on_sc=False),
    )
    def k(table_hbm, idx_hbm, out_hbm, idx_v, rows_v, sem):
        cid = lax.axis_index("c")
        sid = lax.axis_index("s")
        wid = sid * 2 + cid

        def body(j, _):
            b = wid + j * _SC_NW

            @pl.when(b < _SC_NBLK)
            def _():
                base = b * _SC_CHUNK
                pltpu.sync_copy(idx_hbm.at[pl.ds(base, _SC_CHUNK)], idx_v)
                pltpu.async_copy(table_hbm.at[idx_v], rows_v, sem).wait()
                pltpu.sync_copy(rows_v, out_hbm.at[pl.ds(base, _SC_CHUNK), :])

            return 0

        lax.fori_loop(0, _SC_ITERS, body, 0)

    return k(table, idx)


def _lin1_body(x_ref, w_ref, b_ref, o_ref):
    o_ref[...] = (
        jnp.dot(x_ref[...], w_ref[...], preferred_element_type=jnp.float32,
                precision=jax.lax.Precision.HIGHEST)
        + b_ref[...]
    )


def _lin1(x, wt, b):
    return pl.pallas_call(
        _lin1_body,
        grid=(_N // _TN,),
        in_specs=[
            pl.BlockSpec((_TN, x.shape[1]), lambda i: (i, 0)),
            pl.BlockSpec(wt.shape, lambda i: (0, 0)),
            pl.BlockSpec((1, _AF), lambda i: (0, 0)),
        ],
        out_specs=pl.BlockSpec((_TN, _AF), lambda i: (i, 0)),
        out_shape=jax.ShapeDtypeStruct((_N, _AF), jnp.float32),
    )(x, wt, b)


def _gated_body(atom_ref, ga_ref, edge_ref, w_ref, b_ref,
                gated_ref, s1_ref, sq1_ref):
    i = pl.program_id(0)
    te = _TA * _M
    self3 = jnp.broadcast_to(
        atom_ref[...][:, None, :], (_TA, _M, _AF)
    ).reshape(te, _AF)
    total = jnp.concatenate([self3, ga_ref[...], edge_ref[...]], axis=1)
    q = (
        jnp.dot(total, w_ref[...], preferred_element_type=jnp.float32,
                precision=jax.lax.Precision.HIGHEST)
        + b_ref[...]
    )
    q3 = q.reshape(_TA, _M, 2 * _AF)
    gated_ref[...] = q3

    @pl.when(i == 0)
    def _():
        s1_ref[...] = jnp.zeros_like(s1_ref)
        sq1_ref[...] = jnp.zeros_like(sq1_ref)

    s1_ref[...] += jnp.sum(q3, axis=(0, 1))[None, :]
    sq1_ref[...] += jnp.sum(q3 * q3, axis=(0, 1))[None, :]


def _gated(atom, ga, edge, w, b):
    te = _TA * _M
    return pl.pallas_call(
        _gated_body,
        grid=(_GRID,),
        in_specs=[
            pl.BlockSpec((_TA, _AF), lambda i: (i, 0)),
            pl.BlockSpec((te, _AF), lambda i: (i, 0)),
            pl.BlockSpec((te, _NF), lambda i: (i, 0)),
            pl.BlockSpec((2 * _AF + _NF, 2 * _AF), lambda i: (0, 0)),
            pl.BlockSpec((1, 2 * _AF), lambda i: (0, 0)),
        ],
        out_specs=[
            pl.BlockSpec((_TA, _M, 2 * _AF), lambda i: (i, 0, 0)),
            pl.BlockSpec((1, 2 * _AF), lambda i: (0, 0)),
            pl.BlockSpec((1, 2 * _AF), lambda i: (0, 0)),
        ],
        out_shape=[
            jax.ShapeDtypeStruct((_N, _M, 2 * _AF), jnp.float32),
            jax.ShapeDtypeStruct((1, 2 * _AF), jnp.float32),
            jax.ShapeDtypeStruct((1, 2 * _AF), jnp.float32),
        ],
    )(atom, ga, edge, w, b)


def _gate_reduce_body(gated_ref, a1_ref, c1_ref, ns_ref, s2_ref, sq2_ref):
    i = pl.program_id(0)
    q = gated_ref[...] * a1_ref[...][None] + c1_ref[...][None]
    f = jax.nn.sigmoid(q[:, :, :_AF])
    c = jax.nn.softplus(q[:, :, _AF:])
    ns = jnp.sum(f * c, axis=1)
    ns_ref[...] = ns

    @pl.when(i == 0)
    def _():
        s2_ref[...] = jnp.zeros_like(s2_ref)
        sq2_ref[...] = jnp.zeros_like(sq2_ref)

    s2_ref[...] += jnp.sum(ns, axis=0)[None, :]
    sq2_ref[...] += jnp.sum(ns * ns, axis=0)[None, :]


def _gate_reduce(gated, a1, c1):
    return pl.pallas_call(
        _gate_reduce_body,
        grid=(_GRID,),
        in_specs=[
            pl.BlockSpec((_TA, _M, 2 * _AF), lambda i: (i, 0, 0)),
            pl.BlockSpec((1, 2 * _AF), lambda i: (0, 0)),
            pl.BlockSpec((1, 2 * _AF), lambda i: (0, 0)),
        ],
        out_specs=[
            pl.BlockSpec((_TA, _AF), lambda i: (i, 0)),
            pl.BlockSpec((1, _AF), lambda i: (0, 0)),
            pl.BlockSpec((1, _AF), lambda i: (0, 0)),
        ],
        out_shape=[
            jax.ShapeDtypeStruct((_N, _AF), jnp.float32),
            jax.ShapeDtypeStruct((1, _AF), jnp.float32),
            jax.ShapeDtypeStruct((1, _AF), jnp.float32),
        ],
    )(gated, a1, c1)


def _update_body(atom_ref, ns_ref, a2_ref, c2_ref, o_ref):
    o_ref[...] = jax.nn.softplus(
        atom_ref[...] + ns_ref[...] * a2_ref[...] + c2_ref[...]
    )


def _update(atom, ns, a2, c2):
    return pl.pallas_call(
        _update_body,
        grid=(_N // _TN,),
        in_specs=[
            pl.BlockSpec((_TN, _AF), lambda i: (i, 0)),
            pl.BlockSpec((_TN, _AF), lambda i: (i, 0)),
            pl.BlockSpec((1, _AF), lambda i: (0, 0)),
            pl.BlockSpec((1, _AF), lambda i: (0, 0)),
        ],
        out_specs=pl.BlockSpec((_TN, _AF), lambda i: (i, 0)),
        out_shape=jax.ShapeDtypeStruct((_N, _AF), jnp.float32),
    )(atom, ns, a2, c2)


def _update_pool_body(atom_ref, ns_ref, a2_ref, c2_ref, bf_ref,
                      sums_ref, cnt_ref):
    i = pl.program_id(0)
    af = jax.nn.softplus(
        atom_ref[...] + ns_ref[...] * a2_ref[...] + c2_ref[...]
    )
    seg = jax.lax.broadcasted_iota(jnp.int32, (1, _G), 1).astype(jnp.float32)
    oh = (bf_ref[...] == seg).astype(jnp.float32)

    @pl.when(i == 0)
    def _():
        sums_ref[...] = jnp.zeros_like(sums_ref)
        cnt_ref[...] = jnp.zeros_like(cnt_ref)

    sums_ref[...] += jax.lax.dot_general(
        oh, af, (((0,), (0,)), ((), ())), preferred_element_type=jnp.float32,
        precision=jax.lax.Precision.HIGHEST,
    )
    cnt_ref[...] += jnp.sum(oh, axis=0)[None, :]


def _update_pool(atom, ns, a2, c2, batch_f):
    return pl.pallas_call(
        _update_pool_body,
        grid=(_N // _TN,),
        in_specs=[
            pl.BlockSpec((_TN, _AF), lambda i: (i, 0)),
            pl.BlockSpec((_TN, _AF), lambda i: (i, 0)),
            pl.BlockSpec((1, _AF), lambda i: (0, 0)),
            pl.BlockSpec((1, _AF), lambda i: (0, 0)),
            pl.BlockSpec((_TN, 1), lambda i: (i, 0)),
        ],
        out_specs=[
            pl.BlockSpec((_G, _AF), lambda i: (0, 0)),
            pl.BlockSpec((1, _G), lambda i: (0, 0)),
        ],
        out_shape=[
            jax.ShapeDtypeStruct((_G, _AF), jnp.float32),
            jax.ShapeDtypeStruct((1, _G), jnp.float32),
        ],
    )(atom, ns, a2, c2, batch_f)


def _bn_rows(h, g, b):
    mu = jnp.mean(h, axis=0, keepdims=True)
    var = jnp.mean((h - mu) * (h - mu), axis=0, keepdims=True)
    return (h - mu) * jax.lax.rsqrt(var + _EPS) * g + b


def _head_body(sums_ref, invc_ref, l2t_ref, l2b_ref, big_ref, bib_ref,
               c1t_ref, c1b_ref, g1_ref, b1_ref,
               c2t_ref, c2b_ref, g2_ref, b2_ref,
               rt_ref, rb_ref, o_ref):
    pooled = sums_ref[...] * invc_ref[...]
    h = jnp.dot(pooled, l2t_ref[...], preferred_element_type=jnp.float32,
                precision=jax.lax.Precision.HIGHEST) + l2b_ref[...]
    h = jnp.maximum(_bn_rows(h, big_ref[...], bib_ref[...]), 0.0)
    h = jnp.dot(h, c1t_ref[...], preferred_element_type=jnp.float32,
                precision=jax.lax.Precision.HIGHEST) + c1b_ref[...]
    h = jnp.maximum(_bn_rows(h, g1_ref[...], b1_ref[...]), 0.0)
    h = jnp.dot(h, c2t_ref[...], preferred_element_type=jnp.float32,
                precision=jax.lax.Precision.HIGHEST) + c2b_ref[...]
    h = jnp.maximum(_bn_rows(h, g2_ref[...], b2_ref[...]), 0.0)
    o_ref[...] = jnp.dot(h, rt_ref[...], preferred_element_type=jnp.float32,
                precision=jax.lax.Precision.HIGHEST) + rb_ref[...]


def _head(sums, invc, l2t, l2b, big, bib, c1t, c1b, g1, b1,
          c2t, c2b, g2, b2, rt, rb):
    return pl.pallas_call(
        _head_body,
        out_shape=jax.ShapeDtypeStruct((_G, 1), jnp.float32),
    )(sums, invc, l2t, l2b, big, bib, c1t, c1b, g1, b1,
      c2t, c2b, g2, b2, rt, rb)


def _row(v):
    return v.reshape(1, -1).astype(jnp.float32)


def kernel(x, edge_attr, edge_index, batch,
           lin1_w, lin1_b,
           conv_fc_w, conv_fc_b, conv_bn1_g, conv_bn1_b, conv_bn2_g, conv_bn2_b,
           lin2_w, lin2_b, bn_in_g, bn_in_b,
           cl1_w, cl1_b, cbn1_g, cbn1_b,
           cl2_w, cl2_b, cbn2_g, cbn2_b,
           reg_w, reg_b):
    idx = edge_index[1].astype(jnp.int32)
    batch_f = batch.astype(jnp.float32).reshape(_N, 1)

    atom = _lin1(x, lin1_w.T, _row(lin1_b))

    for i in range(_NCONV):
        ga = _sc_gather(atom, idx)
        gated, s1, sq1 = _gated(atom, ga, edge_attr, conv_fc_w[i].T,
                                _row(conv_fc_b[i]))

        mu1 = s1 / float(_E)
        var1 = sq1 / float(_E) - mu1 * mu1
        a1 = _row(conv_bn1_g[i]) * jax.lax.rsqrt(var1 + _EPS)
        c1 = _row(conv_bn1_b[i]) - mu1 * a1

        ns, s2, sq2 = _gate_reduce(gated, a1, c1)

        mu2 = s2 / float(_N)
        var2 = sq2 / float(_N) - mu2 * mu2
        a2 = _row(conv_bn2_g[i]) * jax.lax.rsqrt(var2 + _EPS)
        c2 = _row(conv_bn2_b[i]) - mu2 * a2

        if i < _NCONV - 1:
            atom = _update(atom, ns, a2, c2)
        else:
            sums, cnt = _update_pool(atom, ns, a2, c2, batch_f)

    invc = (1.0 / jnp.maximum(cnt, 1.0)).reshape(_G, 1)
    return _head(sums, invc,
                 lin2_w.T, _row(lin2_b), _row(bn_in_g), _row(bn_in_b),
                 cl1_w.T, _row(cl1_b), _row(cbn1_g), _row(cbn1_b),
                 cl2_w.T, _row(cl2_b), _row(cbn2_g), _row(cbn2_b),
                 reg_w.T, _row(reg_b))
